# baseline (speedup 1.0000x reference)
import functools

import jax
import jax.numpy as jnp
from jax import lax
from jax.experimental import pallas as pl
from jax.experimental.pallas import tpu as pltpu
from jax.experimental.pallas import tpu_sc as plsc

BATCH = 16384
EMBED = 64
HIDDEN = 128
NC = 2
NS = 16
NW = NC * NS
B_PER_W = BATCH // NW
CHUNK = 128
NCHUNK = B_PER_W // CHUNK
BM = 2048


def _sc_gather_body(uid_hbm, iid_hbm, ut_hbm, it_hbm, ou_hbm, oi_hbm,
                    uidx_v, iidx_v, urows_v, irows_v, sem):
    wid = lax.axis_index("s") * NC + lax.axis_index("c")
    base = wid * NCHUNK
    pltpu.sync_copy(uid_hbm.at[pl.ds(base, NCHUNK)], uidx_v)
    pltpu.sync_copy(iid_hbm.at[pl.ds(base, NCHUNK)], iidx_v)
    cps = []
    for j in range(NCHUNK):
        cps.append(pltpu.async_copy(ut_hbm.at[uidx_v.at[j]], urows_v.at[j], sem))
        cps.append(pltpu.async_copy(it_hbm.at[iidx_v.at[j]], irows_v.at[j], sem))
    for c in cps:
        c.wait()
    pltpu.sync_copy(urows_v, ou_hbm.at[pl.ds(base, NCHUNK)])
    pltpu.sync_copy(irows_v, oi_hbm.at[pl.ds(base, NCHUNK)])


_sc_gather = functools.partial(
    pl.kernel,
    mesh=plsc.VectorSubcoreMesh(core_axis_name="c", subcore_axis_name="s"),
    out_type=[
        jax.ShapeDtypeStruct((BATCH // CHUNK, CHUNK, EMBED), jnp.float32),
        jax.ShapeDtypeStruct((BATCH // CHUNK, CHUNK, EMBED), jnp.float32),
    ],
    scratch_types=[
        pltpu.VMEM((NCHUNK, CHUNK), jnp.int32),
        pltpu.VMEM((NCHUNK, CHUNK), jnp.int32),
        pltpu.VMEM((NCHUNK, CHUNK, EMBED), jnp.float32),
        pltpu.VMEM((NCHUNK, CHUNK, EMBED), jnp.float32),
        pltpu.SemaphoreType.DMA,
    ],
    compiler_params=pltpu.CompilerParams(use_tc_tiling_on_sc=False),
)(_sc_gather_body)


def _mlp_body(u_ref, i_ref, w1u_ref, w1i_ref, b1_ref, w2_ref, b2_ref, o_ref):
    h = jnp.dot(u_ref[...], w1u_ref[...], preferred_element_type=jnp.float32)
    h = h + jnp.dot(i_ref[...], w1i_ref[...], preferred_element_type=jnp.float32)
    h = jnp.maximum(h + b1_ref[...], 0.0)
    p = jnp.sum(h * w2_ref[...], axis=1) + b2_ref[0, 0]
    o_ref[...] = jax.nn.sigmoid(p)


def kernel(user_ids, item_ids, user_table, item_table, W1, b1, W2, b2):
    uid = user_ids.astype(jnp.int32).reshape(BATCH // CHUNK, CHUNK)
    iid = item_ids.astype(jnp.int32).reshape(BATCH // CHUNK, CHUNK)
    u_rows, i_rows = _sc_gather(uid, iid, user_table, item_table)
    u_rows = u_rows.reshape(BATCH, EMBED)
    i_rows = i_rows.reshape(BATCH, EMBED)

    w1u = W1[:, :EMBED].T
    w1i = W1[:, EMBED:].T
    b1r = b1.reshape(1, HIDDEN)
    w2r = W2.reshape(1, HIDDEN)
    b2r = b2.reshape(1, 1)

    out = pl.pallas_call(
        _mlp_body,
        grid=(BATCH // BM,),
        in_specs=[
            pl.BlockSpec((BM, EMBED), lambda m: (m, 0)),
            pl.BlockSpec((BM, EMBED), lambda m: (m, 0)),
            pl.BlockSpec((EMBED, HIDDEN), lambda m: (0, 0)),
            pl.BlockSpec((EMBED, HIDDEN), lambda m: (0, 0)),
            pl.BlockSpec((1, HIDDEN), lambda m: (0, 0)),
            pl.BlockSpec((1, HIDDEN), lambda m: (0, 0)),
            pl.BlockSpec((1, 1), lambda m: (0, 0)),
        ],
        out_specs=pl.BlockSpec((BM,), lambda m: (m,)),
        out_shape=jax.ShapeDtypeStruct((BATCH,), jnp.float32),
    )(u_rows, i_rows, w1u, w1i, b1r, w2r, b2r)
    return out


# speedup vs baseline: 1.7227x; 1.7227x over previous
import functools

import jax
import jax.numpy as jnp
from jax import lax
from jax.experimental import pallas as pl
from jax.experimental.pallas import tpu as pltpu
from jax.experimental.pallas import tpu_sc as plsc

BATCH = 16384
EMBED = 64
HIDDEN = 128
LANES = 128
NC = 2
NS = 16
NW = NC * NS
B_PER_W = BATCH // NW
BM = 2048


def _gather_table(tbl_t, ids_v, rows_v, col0, col1, sem0, sem1):
    cols = (col0, col1)
    sems = (sem0, sem1)
    NG = B_PER_W // 16

    def fire(idval, buf, sem):
        off = pl.multiple_of((idval >> 7) * LANES, LANES)
        pltpu.async_copy(tbl_t.at[:, pl.ds(off, LANES)], buf, sem)

    def drain(buf, sem):
        pltpu.make_async_copy(tbl_t.at[:, pl.ds(0, LANES)], buf, sem).wait()

    fire(ids_v[pl.ds(0, 16)][0], col0, sem0)

    def body(G, carry):
        g16 = pl.multiple_of(G * 16, 16)
        idv = ids_v[pl.ds(g16, 16)]
        nidv = ids_v[pl.ds(g16 + 16, 16)]
        for k in range(16):
            pk, nk = k % 2, (k + 1) % 2
            if k < 15:
                fire(idv[k + 1], cols[nk], sems[nk])
            else:
                @pl.when(G + 1 < NG)
                def _():
                    fire(nidv[0], cols[nk], sems[nk])
            drain(cols[pk], sems[pk])
            lane = jnp.bitwise_and(idv[k], LANES - 1)
            lv = jnp.full((16,), lane, dtype=jnp.int32)
            for g in range(EMBED // 16):
                ev = lax.iota(jnp.int32, 16) + (16 * g)
                vals = plsc.load_gather(cols[pk], [ev, lv])
                rows_v[g16 + k, pl.ds(16 * g, 16)] = vals
        return carry

    lax.fori_loop(0, NG, body, 0, unroll=False)


def _sc_gather_body(uid_hbm, iid_hbm, ut_hbm, it_hbm, ou_hbm, oi_hbm,
                    uid_s, iid_s, rows_v, col0, col1, sem0, sem1):
    wid = lax.axis_index("s") * NC + lax.axis_index("c")
    base = wid * B_PER_W
    pltpu.sync_copy(uid_hbm.at[pl.ds(base, B_PER_W)], uid_s.at[pl.ds(0, B_PER_W)])
    pltpu.sync_copy(iid_hbm.at[pl.ds(base, B_PER_W)], iid_s.at[pl.ds(0, B_PER_W)])

    _gather_table(ut_hbm, uid_s, rows_v, col0, col1, sem0, sem1)
    pltpu.sync_copy(rows_v, ou_hbm.at[pl.ds(base, B_PER_W)])

    _gather_table(it_hbm, iid_s, rows_v, col0, col1, sem0, sem1)
    pltpu.sync_copy(rows_v, oi_hbm.at[pl.ds(base, B_PER_W)])


_sc_gather = functools.partial(
    pl.kernel,
    mesh=plsc.VectorSubcoreMesh(core_axis_name="c", subcore_axis_name="s"),
    out_type=[
        jax.ShapeDtypeStruct((BATCH, EMBED), jnp.float32),
        jax.ShapeDtypeStruct((BATCH, EMBED), jnp.float32),
    ],
    scratch_types=[
        pltpu.VMEM((B_PER_W + 16,), jnp.int32),
        pltpu.VMEM((B_PER_W + 16,), jnp.int32),
        pltpu.VMEM((B_PER_W, EMBED), jnp.float32),
        pltpu.VMEM((EMBED, LANES), jnp.float32),
        pltpu.VMEM((EMBED, LANES), jnp.float32),
        pltpu.SemaphoreType.DMA,
        pltpu.SemaphoreType.DMA,
    ],
    compiler_params=pltpu.CompilerParams(needs_layout_passes=False),
)(_sc_gather_body)


def _mlp_body(u_ref, i_ref, w1u_ref, w1i_ref, b1_ref, w2_ref, b2_ref, o_ref):
    h = jnp.dot(u_ref[...], w1u_ref[...], preferred_element_type=jnp.float32)
    h = h + jnp.dot(i_ref[...], w1i_ref[...], preferred_element_type=jnp.float32)
    h = jnp.maximum(h + b1_ref[...], 0.0)
    p = jnp.sum(h * w2_ref[...], axis=1) + b2_ref[0, 0]
    o_ref[...] = jax.nn.sigmoid(p)


def kernel(user_ids, item_ids, user_table, item_table, W1, b1, W2, b2):
    uid = user_ids.astype(jnp.int32)
    iid = item_ids.astype(jnp.int32)
    ut_t = user_table.T
    it_t = item_table.T
    u_rows, i_rows = _sc_gather(uid, iid, ut_t, it_t)

    w1u = W1[:, :EMBED].T
    w1i = W1[:, EMBED:].T
    b1r = b1.reshape(1, HIDDEN)
    w2r = W2.reshape(1, HIDDEN)
    b2r = b2.reshape(1, 1)

    out = pl.pallas_call(
        _mlp_body,
        grid=(BATCH // BM,),
        in_specs=[
            pl.BlockSpec((BM, EMBED), lambda m: (m, 0)),
            pl.BlockSpec((BM, EMBED), lambda m: (m, 0)),
            pl.BlockSpec((EMBED, HIDDEN), lambda m: (0, 0)),
            pl.BlockSpec((EMBED, HIDDEN), lambda m: (0, 0)),
            pl.BlockSpec((1, HIDDEN), lambda m: (0, 0)),
            pl.BlockSpec((1, HIDDEN), lambda m: (0, 0)),
            pl.BlockSpec((1, 1), lambda m: (0, 0)),
        ],
        out_specs=pl.BlockSpec((BM,), lambda m: (m,)),
        out_shape=jax.ShapeDtypeStruct((BATCH,), jnp.float32),
    )(u_rows, i_rows, w1u, w1i, b1r, w2r, b2r)
    return out


# speedup vs baseline: 2.3502x; 1.3642x over previous
import functools

import jax
import jax.numpy as jnp
from jax import lax
from jax.experimental import pallas as pl
from jax.experimental.pallas import tpu as pltpu
from jax.experimental.pallas import tpu_sc as plsc

BATCH = 16384
EMBED = 64
HIDDEN = 128
LANES = 128
NC = 2
NS = 16
NW = NC * NS
B_PER_W = BATCH // NW
BM = 2048


DEPTH = 4


def _gather_table(tbl_t, ids_v, rows_v, cols, sems):
    NG = B_PER_W // 16

    def fire(idval, buf, sem):
        off = pl.multiple_of((idval >> 7) * LANES, LANES)
        pltpu.async_copy(tbl_t.at[:, pl.ds(off, LANES)], buf, sem)

    def drain(buf, sem):
        pltpu.make_async_copy(tbl_t.at[:, pl.ds(0, LANES)], buf, sem).wait()

    idv0 = ids_v[pl.ds(0, 16)]
    for k in range(DEPTH - 1):
        fire(idv0[k], cols[k], sems[k])

    def body(G, carry):
        g16 = pl.multiple_of(G * 16, 16)
        idv = ids_v[pl.ds(g16, 16)]
        nidv = ids_v[pl.ds(g16 + 16, 16)]
        for k in range(16):
            pk, nk = k % DEPTH, (k + DEPTH - 1) % DEPTH
            nxt = idv[k + DEPTH - 1] if k < 16 - (DEPTH - 1) else nidv[k - 17 + DEPTH]
            if k < 16 - (DEPTH - 1):
                fire(nxt, cols[nk], sems[nk])
            else:
                @pl.when(G + 1 < NG)
                def _():
                    fire(nxt, cols[nk], sems[nk])
            drain(cols[pk], sems[pk])
            lane = jnp.bitwise_and(idv[k], LANES - 1)
            lv = jnp.full((16,), lane, dtype=jnp.int32)
            for g in range(EMBED // 16):
                ev = lax.iota(jnp.int32, 16) + (16 * g)
                vals = plsc.load_gather(cols[pk], [ev, lv])
                rows_v[g16 + k, pl.ds(16 * g, 16)] = vals
        return carry

    lax.fori_loop(0, NG, body, 0, unroll=False)


def _sc_gather_body(uid_hbm, iid_hbm, ut_hbm, it_hbm, ou_hbm, oi_hbm,
                    uid_s, iid_s, rows_v, col0, col1, col2, col3,
                    sem0, sem1, sem2, sem3):
    wid = lax.axis_index("s") * NC + lax.axis_index("c")
    base = wid * B_PER_W
    pltpu.sync_copy(uid_hbm.at[pl.ds(base, B_PER_W)], uid_s.at[pl.ds(0, B_PER_W)])
    pltpu.sync_copy(iid_hbm.at[pl.ds(base, B_PER_W)], iid_s.at[pl.ds(0, B_PER_W)])

    cols = (col0, col1, col2, col3)
    sems = (sem0, sem1, sem2, sem3)
    _gather_table(ut_hbm, uid_s, rows_v, cols, sems)
    pltpu.sync_copy(rows_v, ou_hbm.at[pl.ds(base, B_PER_W)])

    _gather_table(it_hbm, iid_s, rows_v, cols, sems)
    pltpu.sync_copy(rows_v, oi_hbm.at[pl.ds(base, B_PER_W)])


_sc_gather = functools.partial(
    pl.kernel,
    mesh=plsc.VectorSubcoreMesh(core_axis_name="c", subcore_axis_name="s"),
    out_type=[
        jax.ShapeDtypeStruct((BATCH, EMBED), jnp.float32),
        jax.ShapeDtypeStruct((BATCH, EMBED), jnp.float32),
    ],
    scratch_types=[
        pltpu.VMEM((B_PER_W + 16,), jnp.int32),
        pltpu.VMEM((B_PER_W + 16,), jnp.int32),
        pltpu.VMEM((B_PER_W, EMBED), jnp.float32),
        pltpu.VMEM((EMBED, LANES), jnp.float32),
        pltpu.VMEM((EMBED, LANES), jnp.float32),
        pltpu.VMEM((EMBED, LANES), jnp.float32),
        pltpu.VMEM((EMBED, LANES), jnp.float32),
        pltpu.SemaphoreType.DMA,
        pltpu.SemaphoreType.DMA,
        pltpu.SemaphoreType.DMA,
        pltpu.SemaphoreType.DMA,
    ],
    compiler_params=pltpu.CompilerParams(needs_layout_passes=False),
)(_sc_gather_body)


def _mlp_body(u_ref, i_ref, w1u_ref, w1i_ref, b1_ref, w2_ref, b2_ref, o_ref):
    h = jnp.dot(u_ref[...], w1u_ref[...], preferred_element_type=jnp.float32)
    h = h + jnp.dot(i_ref[...], w1i_ref[...], preferred_element_type=jnp.float32)
    h = jnp.maximum(h + b1_ref[...], 0.0)
    p = jnp.sum(h * w2_ref[...], axis=1) + b2_ref[0, 0]
    o_ref[...] = jax.nn.sigmoid(p)


def kernel(user_ids, item_ids, user_table, item_table, W1, b1, W2, b2):
    uid = user_ids.astype(jnp.int32)
    iid = item_ids.astype(jnp.int32)
    ut_t = user_table.T
    it_t = item_table.T
    u_rows, i_rows = _sc_gather(uid, iid, ut_t, it_t)

    w1u = W1[:, :EMBED].T
    w1i = W1[:, EMBED:].T
    b1r = b1.reshape(1, HIDDEN)
    w2r = W2.reshape(1, HIDDEN)
    b2r = b2.reshape(1, 1)

    out = pl.pallas_call(
        _mlp_body,
        grid=(BATCH // BM,),
        in_specs=[
            pl.BlockSpec((BM, EMBED), lambda m: (m, 0)),
            pl.BlockSpec((BM, EMBED), lambda m: (m, 0)),
            pl.BlockSpec((EMBED, HIDDEN), lambda m: (0, 0)),
            pl.BlockSpec((EMBED, HIDDEN), lambda m: (0, 0)),
            pl.BlockSpec((1, HIDDEN), lambda m: (0, 0)),
            pl.BlockSpec((1, HIDDEN), lambda m: (0, 0)),
            pl.BlockSpec((1, 1), lambda m: (0, 0)),
        ],
        out_specs=pl.BlockSpec((BM,), lambda m: (m,)),
        out_shape=jax.ShapeDtypeStruct((BATCH,), jnp.float32),
    )(u_rows, i_rows, w1u, w1i, b1r, w2r, b2r)
    return out
